# baseline (speedup 1.0000x reference)
import functools

import jax
import jax.numpy as jnp
from jax import lax
from jax.experimental import pallas as pl
from jax.experimental.pallas import tpu as pltpu
from jax.experimental.pallas import tpu_sc as plsc

_NUM_CORES = 2
_NUM_SUBCORES = 16
_NUM_WORKERS = _NUM_CORES * _NUM_SUBCORES
_LANES = 16
_IDX_CHUNK = 128


def _mf_forward(user_idx, item_idx, W, H):
    B = user_idx.shape[0]
    K = W.shape[1]
    b_per_w = B // _NUM_WORKERS
    n_chunks = b_per_w // _IDX_CHUNK
    groups = b_per_w // _LANES

    mesh = plsc.VectorSubcoreMesh(core_axis_name="c", subcore_axis_name="s")

    @functools.partial(
        pl.kernel,
        mesh=mesh,
        out_type=jax.ShapeDtypeStruct((B,), jnp.float32),
        scratch_types=[
            pltpu.VMEM((n_chunks, _IDX_CHUNK), jnp.int32),
            pltpu.VMEM((n_chunks, _IDX_CHUNK), jnp.int32),
            pltpu.VMEM((b_per_w, K), jnp.float32),
            pltpu.VMEM((b_per_w, K), jnp.float32),
            pltpu.VMEM((b_per_w,), jnp.float32),
            pltpu.SemaphoreType.DMA,
        ],
        compiler_params=pltpu.CompilerParams(
            needs_layout_passes=False, use_tc_tiling_on_sc=False),
    )
    def mf_kernel(uidx_hbm, vidx_hbm, w_hbm, h_hbm, out_hbm,
                  uidx_v, vidx_v, u_rows, v_rows, out_v, sem):
        wid = lax.axis_index("s") * _NUM_CORES + lax.axis_index("c")
        base = wid * b_per_w

        copies = []
        for c in range(n_chunks):
            off = base + c * _IDX_CHUNK
            pltpu.sync_copy(uidx_hbm.at[pl.ds(off, _IDX_CHUNK)], uidx_v.at[c])
            pltpu.sync_copy(vidx_hbm.at[pl.ds(off, _IDX_CHUNK)], vidx_v.at[c])
            dst = pl.ds(c * _IDX_CHUNK, _IDX_CHUNK)
            copies.append(pltpu.async_copy(w_hbm.at[uidx_v.at[c]],
                                           u_rows.at[dst], sem))
            copies.append(pltpu.async_copy(h_hbm.at[vidx_v.at[c]],
                                           v_rows.at[dst], sem))
        for cp in copies:
            cp.wait()

        def group_body(g, carry):
            rows = g * _LANES + lax.iota(jnp.int32, _LANES)
            acc = jnp.zeros((_LANES,), jnp.float32)
            for kk in range(K):
                col = jnp.full((_LANES,), kk, jnp.int32)
                u = plsc.load_gather(u_rows, [rows, col])
                v = plsc.load_gather(v_rows, [rows, col])
                acc = acc + u * v
            out_v[pl.ds(g * _LANES, _LANES)] = 1.0 / (1.0 + jnp.exp(-acc))
            return carry

        lax.fori_loop(0, groups, group_body, 0)
        pltpu.sync_copy(out_v, out_hbm.at[pl.ds(base, b_per_w)])

    return mf_kernel(user_idx, item_idx, W, H)


def kernel(x, W, H):
    user_idx = x[:, 0].astype(jnp.int32)
    item_idx = x[:, 1].astype(jnp.int32)
    return _mf_forward(user_idx, item_idx, W, H)


# speedup vs baseline: 1.5309x; 1.5309x over previous
import functools

import jax
import jax.numpy as jnp
from jax import lax
from jax.experimental import pallas as pl
from jax.experimental.pallas import tpu as pltpu
from jax.experimental.pallas import tpu_sc as plsc

_NUM_CORES = 2
_NUM_SUBCORES = 16
_NUM_WORKERS = _NUM_CORES * _NUM_SUBCORES
_LANES = 16
_CHUNK = 256


def _mf_forward(user_idx, item_idx, W, H):
    B = user_idx.shape[0]
    K = W.shape[1]
    b_per_w = B // _NUM_WORKERS
    n_chunks = b_per_w // _CHUNK
    groups = _CHUNK // _LANES

    mesh = plsc.VectorSubcoreMesh(core_axis_name="c", subcore_axis_name="s")

    @functools.partial(
        pl.kernel,
        mesh=mesh,
        out_type=jax.ShapeDtypeStruct((B,), jnp.float32),
        scratch_types=[
            pltpu.VMEM((b_per_w,), jnp.int32),
            pltpu.VMEM((b_per_w,), jnp.int32),
            pltpu.VMEM((_CHUNK, K), jnp.float32),
            pltpu.VMEM((_CHUNK, K), jnp.float32),
            pltpu.VMEM((b_per_w,), jnp.float32),
            pltpu.SemaphoreType.DMA,
        ],
        compiler_params=pltpu.CompilerParams(needs_layout_passes=False),
    )
    def mf_kernel(uidx_hbm, vidx_hbm, w_hbm, h_hbm, out_hbm,
                  uidx_v, vidx_v, u_rows, v_rows, out_v, sem):
        wid = lax.axis_index("s") * _NUM_CORES + lax.axis_index("c")
        base = wid * b_per_w

        pltpu.sync_copy(uidx_hbm.at[pl.ds(base, b_per_w)], uidx_v)
        pltpu.sync_copy(vidx_hbm.at[pl.ds(base, b_per_w)], vidx_v)

        lane = lax.iota(jnp.int32, _LANES)

        def chunk_body(c, carry):
            cbase = c * _CHUNK

            def fetch_body(g, inner):
                u16 = uidx_v[pl.ds(cbase + g * _LANES, _LANES)]
                v16 = vidx_v[pl.ds(cbase + g * _LANES, _LANES)]
                for j in range(_LANES):
                    dst = g * _LANES + j
                    pltpu.async_copy(w_hbm.at[u16[j]], u_rows.at[dst], sem)
                    pltpu.async_copy(h_hbm.at[v16[j]], v_rows.at[dst], sem)
                return inner

            lax.fori_loop(0, _CHUNK // _LANES, fetch_body, 0)
            pltpu.make_async_copy(w_hbm.at[pl.ds(0, _CHUNK)],
                                  u_rows, sem).wait()
            pltpu.make_async_copy(w_hbm.at[pl.ds(0, _CHUNK)],
                                  v_rows, sem).wait()

            def group_body(g, inner):
                rows = g * _LANES + lane
                acc = jnp.zeros((_LANES,), jnp.float32)
                for kk in range(K):
                    col = (lane + kk) & (K - 1)
                    u = plsc.load_gather(u_rows, [rows, col])
                    v = plsc.load_gather(v_rows, [rows, col])
                    acc = acc + u * v
                out_v[pl.ds(cbase + g * _LANES, _LANES)] = (
                    1.0 / (1.0 + jnp.exp(-acc)))
                return inner

            lax.fori_loop(0, groups, group_body, 0)
            return carry

        lax.fori_loop(0, n_chunks, chunk_body, 0)
        pltpu.sync_copy(out_v, out_hbm.at[pl.ds(base, b_per_w)])

    return mf_kernel(user_idx, item_idx, W, H)


def kernel(x, W, H):
    user_idx = x[:, 0].astype(jnp.int32)
    item_idx = x[:, 1].astype(jnp.int32)
    return _mf_forward(user_idx, item_idx, W, H)
